# baseline (speedup 1.0000x reference)
import jax
import jax.numpy as jnp
from jax.experimental import pallas as pl
from jax.experimental.pallas import tpu as pltpu

N_E = 1024
E_DIM = 64
BETA = 1.0
GAMMA = 0.99
BLOCK = 1024


def _vq_kernel(z_ref, emb_ref,
               oneh_ref, zq_ref, idx_ref, newemb_ref, loss_ref, perp_ref,
               sum_acc, loss_acc, e2_s):
    i = pl.program_id(0)
    nsteps = pl.num_programs(0)
    emb = emb_ref[...]

    @pl.when(i == 0)
    def _init():
        sum_acc[...] = jnp.zeros_like(sum_acc)
        loss_acc[...] = jnp.zeros_like(loss_acc)
        e2_s[...] = jnp.sum(jnp.transpose(emb * emb), axis=0, keepdims=True)

    z3 = z_ref[...].reshape(BLOCK, 2, E_DIM // 2)
    zb = jnp.concatenate([z3[:, 0, :], z3[:, 1, :]], axis=1)

    dot = jnp.dot(zb, emb.T, preferred_element_type=jnp.float32)
    z2 = jnp.sum(zb ** 2, axis=1, keepdims=True)
    d = (z2 + e2_s[...]) - 2.0 * dot

    cols = jax.lax.broadcasted_iota(jnp.int32, (BLOCK, N_E), 1)
    dmin = jnp.min(d, axis=1, keepdims=True)
    idx = jnp.min(jnp.where(d == dmin, cols, N_E), axis=1).astype(jnp.int32)

    oneh = (cols == idx[:, None]).astype(jnp.float32)
    oneh_ref[...] = oneh
    idx_ref[...] = idx.reshape(1, 1, BLOCK)

    zq = jnp.dot(oneh, emb, preferred_element_type=jnp.float32)
    zq_st = zb + (zq - zb)
    zq2 = jnp.stack([zq_st[:, :E_DIM // 2], zq_st[:, E_DIM // 2:]], axis=1)
    zq_ref[...] = zq2.reshape(zq_ref.shape)

    aug = jnp.concatenate([zb, jnp.ones((BLOCK, 1), jnp.float32)], axis=1)
    sum_acc[...] += jax.lax.dot_general(
        oneh, aug, (((0,), (0,)), ((), ())),
        preferred_element_type=jnp.float32)
    diff = zq - zb
    loss_acc[...] += jnp.sum(diff * diff, axis=(0, 1), keepdims=True)

    @pl.when(i == nsteps - 1)
    def _fin():
        total_rows = nsteps * BLOCK
        cnt = sum_acc[:, E_DIM:E_DIM + 1]
        n_col = GAMMA + cnt * (1.0 - GAMMA)
        m_mat = GAMMA + sum_acc[:, :E_DIM] * (1.0 - GAMMA)
        newemb_ref[...] = m_mat / n_col
        e_mean = cnt * (1.0 / total_rows)
        ent = jnp.sum(e_mean * jnp.log(e_mean + 1e-10), axis=(0, 1), keepdims=True)
        perp_ref[...] = jnp.exp(-ent)
        loss_ref[...] = loss_acc[...] * (1.0 / (total_rows * E_DIM))


def kernel(z, batch_size, n_train, embedding_weight):
    B, C, H, W = z.shape
    rows = (B * C * H * W) // E_DIM
    grid = rows // BLOCK

    out_shape = [
        jax.ShapeDtypeStruct((rows, N_E), jnp.float32),
        jax.ShapeDtypeStruct((B, C, H, W), jnp.float32),
        jax.ShapeDtypeStruct((grid, 1, BLOCK), jnp.int32),
        jax.ShapeDtypeStruct((N_E, E_DIM), jnp.float32),
        jax.ShapeDtypeStruct((1, 1), jnp.float32),
        jax.ShapeDtypeStruct((1, 1), jnp.float32),
    ]
    out_specs = [
        pl.BlockSpec((BLOCK, N_E), lambda i: (i, 0)),
        pl.BlockSpec((1, C, H, W), lambda i: (i, 0, 0, 0)),
        pl.BlockSpec((1, 1, BLOCK), lambda i: (i, 0, 0)),
        pl.BlockSpec((N_E, E_DIM), lambda i: (0, 0)),
        pl.BlockSpec((1, 1), lambda i: (0, 0)),
        pl.BlockSpec((1, 1), lambda i: (0, 0)),
    ]
    oneh, zq, idx3, newemb, loss, perp = pl.pallas_call(
        _vq_kernel,
        grid=(grid,),
        in_specs=[
            pl.BlockSpec((1, C, H, W), lambda i: (i, 0, 0, 0)),
            pl.BlockSpec((N_E, E_DIM), lambda i: (0, 0)),
        ],
        out_specs=out_specs,
        out_shape=out_shape,
        scratch_shapes=[
            pltpu.VMEM((N_E, E_DIM + 1), jnp.float32),
            pltpu.VMEM((1, 1), jnp.float32),
            pltpu.VMEM((1, N_E), jnp.float32),
        ],
    )(z, embedding_weight)

    loss_s = loss[0, 0]
    return (loss_s, BETA * loss_s, zq, perp[0, 0],
            oneh, idx3.reshape(-1)[:, None], newemb)


# speedup vs baseline: 1.1432x; 1.1432x over previous
import jax
import jax.numpy as jnp
from jax.experimental import pallas as pl
from jax.experimental.pallas import tpu as pltpu

N_E = 1024
E_DIM = 64
BETA = 1.0
GAMMA = 0.99
BLOCK = 2048


def _vq_kernel(zf_ref, emb_ref,
               oneh_ref, zq_ref, idx_ref, newemb_ref, loss_ref, perp_ref,
               sum_acc, loss_acc, e2_s):
    i = pl.program_id(0)
    nsteps = pl.num_programs(0)
    emb = emb_ref[...]

    @pl.when(i == 0)
    def _init():
        sum_acc[...] = jnp.zeros_like(sum_acc)
        loss_acc[...] = jnp.zeros_like(loss_acc)
        e2_s[...] = jnp.sum(jnp.transpose(emb * emb), axis=0, keepdims=True)

    zb = zf_ref[...]

    dot = jnp.dot(zb, emb.T, preferred_element_type=jnp.float32)
    z2 = jnp.sum(zb ** 2, axis=1, keepdims=True)
    d = (z2 + e2_s[...]) - 2.0 * dot

    cols = jax.lax.broadcasted_iota(jnp.int32, (BLOCK, N_E), 1)
    dmin = jnp.min(d, axis=1, keepdims=True)
    idx = jnp.min(jnp.where(d == dmin, cols, N_E), axis=1).astype(jnp.int32)

    oneh = (cols == idx[:, None]).astype(jnp.float32)
    oneh_ref[...] = oneh
    idx_ref[...] = idx.reshape(1, 1, BLOCK)

    zq = jnp.dot(oneh, emb, preferred_element_type=jnp.float32)
    zq_ref[...] = zb + (zq - zb)

    aug = jnp.concatenate([zb, jnp.ones((BLOCK, 1), jnp.float32)], axis=1)
    sum_acc[...] += jax.lax.dot_general(
        oneh, aug, (((0,), (0,)), ((), ())),
        preferred_element_type=jnp.float32)
    diff = zq - zb
    loss_acc[...] += jnp.sum(diff * diff, axis=(0, 1), keepdims=True)

    @pl.when(i == nsteps - 1)
    def _fin():
        total_rows = nsteps * BLOCK
        cnt = sum_acc[:, E_DIM:E_DIM + 1]
        n_col = GAMMA + cnt * (1.0 - GAMMA)
        m_mat = GAMMA + sum_acc[:, :E_DIM] * (1.0 - GAMMA)
        newemb_ref[...] = m_mat / n_col
        e_mean = cnt * (1.0 / total_rows)
        ent = jnp.sum(e_mean * jnp.log(e_mean + 1e-10), axis=(0, 1), keepdims=True)
        perp_ref[...] = jnp.exp(-ent)
        loss_ref[...] = loss_acc[...] * (1.0 / (total_rows * E_DIM))


def kernel(z, batch_size, n_train, embedding_weight):
    zf = z.reshape(-1, E_DIM)
    rows = zf.shape[0]
    grid = rows // BLOCK

    out_shape = [
        jax.ShapeDtypeStruct((rows, N_E), jnp.float32),
        jax.ShapeDtypeStruct((rows, E_DIM), jnp.float32),
        jax.ShapeDtypeStruct((grid, 1, BLOCK), jnp.int32),
        jax.ShapeDtypeStruct((N_E, E_DIM), jnp.float32),
        jax.ShapeDtypeStruct((1, 1), jnp.float32),
        jax.ShapeDtypeStruct((1, 1), jnp.float32),
    ]
    out_specs = [
        pl.BlockSpec((BLOCK, N_E), lambda i: (i, 0)),
        pl.BlockSpec((BLOCK, E_DIM), lambda i: (i, 0)),
        pl.BlockSpec((1, 1, BLOCK), lambda i: (i, 0, 0)),
        pl.BlockSpec((N_E, E_DIM), lambda i: (0, 0)),
        pl.BlockSpec((1, 1), lambda i: (0, 0)),
        pl.BlockSpec((1, 1), lambda i: (0, 0)),
    ]
    oneh, zq, idx3, newemb, loss, perp = pl.pallas_call(
        _vq_kernel,
        grid=(grid,),
        in_specs=[
            pl.BlockSpec((BLOCK, E_DIM), lambda i: (i, 0)),
            pl.BlockSpec((N_E, E_DIM), lambda i: (0, 0)),
        ],
        out_specs=out_specs,
        out_shape=out_shape,
        scratch_shapes=[
            pltpu.VMEM((N_E, E_DIM + 1), jnp.float32),
            pltpu.VMEM((1, 1), jnp.float32),
            pltpu.VMEM((1, N_E), jnp.float32),
        ],
    )(zf, embedding_weight)

    loss_s = loss[0, 0]
    return (loss_s, BETA * loss_s, zq.reshape(z.shape), perp[0, 0],
            oneh, idx3.reshape(-1)[:, None], newemb)
